# baseline (speedup 1.0000x reference)
import functools

import jax
import jax.numpy as jnp
from jax.experimental import pallas as pl
from jax.experimental.pallas import tpu as pltpu
from jax.experimental.pallas import tpu_sc as plsc

NC = 2
NS = 16
NW = NC * NS
LANES = 16
CHUNK = 128
NBUF = 2
DEG_GRP = 8


def _padded_rows(n_nodes):
  per = -(-n_nodes // (NS * 8)) * 8
  return per, per * NS


def _zero_shared_rows(stage_v, sh, rbase, rows, width):
  @pl.loop(0, CHUNK)
  def _(i):
    @pl.loop(0, width, step=LANES)
    def _(jj):
      stage_v[i, pl.ds(jj, LANES)] = jnp.zeros((LANES,), jnp.float32)

  @pl.loop(0, rows // CHUNK)
  def _(t):
    pltpu.sync_copy(stage_v, sh.at[pl.ds(rbase + t * CHUNK, CHUNK)])
  rem = rows % CHUNK
  if rem:
    pltpu.sync_copy(stage_v.at[pl.ds(0, rem)],
                    sh.at[pl.ds(rbase + (rows // CHUNK) * CHUNK, rem)])


def _make_degree_kernel(n_nodes, n_chunks):
  k_tile = n_chunks // NW
  rows_tile, np_rows = _padded_rows(n_nodes)

  mesh = plsc.VectorSubcoreMesh(core_axis_name="c", subcore_axis_name="s",
                                num_cores=NC, num_subcores=NS)

  @functools.partial(
      pl.kernel,
      out_type=jax.ShapeDtypeStruct((NC, np_rows, LANES), jnp.float32),
      mesh=mesh,
      scratch_types=[
          pltpu.VMEM((k_tile, CHUNK), jnp.int32),
          pltpu.VMEM((CHUNK, LANES), jnp.float32),
          pltpu.VMEM_SHARED((np_rows, LANES), jnp.float32),
      ] + [pltpu.SemaphoreType.DMA] * DEG_GRP,
  )
  def k(dst_hbm, degp_hbm, dst_v, ones_v, deg_sh, *sems):
    c = jax.lax.axis_index("c")
    s = jax.lax.axis_index("s")
    wid = c * NS + s
    rbase = s * rows_tile

    pltpu.sync_copy(dst_hbm.at[pl.ds(wid * k_tile, k_tile)], dst_v)
    _zero_shared_rows(ones_v, deg_sh, rbase, rows_tile, LANES)

    @pl.loop(0, CHUNK)
    def _(i):
      ones_v[i, :] = jnp.ones((LANES,), jnp.float32)

    plsc.subcore_barrier()

    @pl.loop(0, k_tile, step=DEG_GRP)
    def _(t):
      descs = []
      for b in range(DEG_GRP):
        descs.append(pltpu.async_copy(
            ones_v, deg_sh.at[dst_v.at[t + b]], sems[b], add=True))
      for d_ in descs:
        d_.wait()

    plsc.subcore_barrier()
    pltpu.sync_copy(deg_sh.at[pl.ds(rbase, rows_tile)],
                    degp_hbm.at[c].at[pl.ds(rbase, rows_tile)])

  return k


def _make_scatter_kernel(n_nodes, n_chunks, d):
  k_tile = n_chunks // NW
  rows_tile, np_rows = _padded_rows(n_nodes)
  ngroups = k_tile // NBUF
  assert k_tile % NBUF == 0

  mesh = plsc.VectorSubcoreMesh(core_axis_name="c", subcore_axis_name="s",
                                num_cores=NC, num_subcores=NS)

  rows_scratch = [pltpu.VMEM((CHUNK, d), jnp.float32) for _ in range(NBUF)]
  dstb_scratch = [pltpu.VMEM((1, CHUNK), jnp.int32) for _ in range(NBUF)]

  @functools.partial(
      pl.kernel,
      out_type=jax.ShapeDtypeStruct((NC, np_rows, d), jnp.float32),
      mesh=mesh,
      scratch_types=[
          pltpu.VMEM((k_tile, CHUNK), jnp.int32),
          pltpu.VMEM_SHARED((np_rows, d), jnp.float32),
      ] + rows_scratch + dstb_scratch
        + [pltpu.SemaphoreType.DMA] * (2 * NBUF),
  )
  def k(u_hbm, src_hbm, dst_hbm, vp_hbm, src_v, v_sh, *rest):
    rows = rest[:NBUF]
    dstb = rest[NBUF:2 * NBUF]
    gsem = rest[2 * NBUF:3 * NBUF]
    dsem = rest[3 * NBUF:4 * NBUF]
    c = jax.lax.axis_index("c")
    s = jax.lax.axis_index("s")
    wid = c * NS + s
    rbase = s * rows_tile
    crow = wid * k_tile

    pltpu.sync_copy(src_hbm.at[pl.ds(crow, k_tile)], src_v)

    _zero_shared_rows(rows[0], v_sh, rbase, rows_tile, d)

    for b in range(NBUF):
      pltpu.async_copy(dst_hbm.at[crow + b], dstb[b], dsem[b])
      pltpu.async_copy(u_hbm.at[src_v.at[b]], rows[b], gsem[b])

    plsc.subcore_barrier()

    def wait_idx(b):
      pltpu.make_async_copy(dst_hbm.at[0], dstb[b], dsem[b]).wait()

    def wait_rows(b):
      pltpu.make_async_copy(u_hbm.at[pl.ds(0, CHUNK)], rows[b], gsem[b]).wait()

    @pl.loop(0, ngroups - 1)
    def _(t):
      for b in range(NBUF):
        j = t * NBUF + b
        wait_idx(b)
        wait_rows(b)
        pltpu.sync_copy(rows[b], v_sh.at[dstb[b].at[0]], add=True)
        pltpu.async_copy(dst_hbm.at[crow + j + NBUF], dstb[b], dsem[b])
        pltpu.async_copy(u_hbm.at[src_v.at[j + NBUF]], rows[b], gsem[b])

    for b in range(NBUF):
      wait_idx(b)
      wait_rows(b)
      pltpu.sync_copy(rows[b], v_sh.at[dstb[b].at[0]], add=True)

    plsc.subcore_barrier()
    pltpu.sync_copy(v_sh.at[pl.ds(rbase, rows_tile)],
                    vp_hbm.at[c].at[pl.ds(rbase, rows_tile)])

  return k


def _scale_body(n, np_rows, degp_ref, x_ref, u_ref):
  deg = degp_ref[0, 0:n, 0:1] + degp_ref[1, 0:n, 0:1] + 1.0
  u_ref[0:n, :] = x_ref[...] * jax.lax.rsqrt(deg)
  u_ref[n:np_rows, :] = jnp.zeros((np_rows - n, x_ref.shape[1]), jnp.float32)


def _combine_body(n, vp_ref, u_ref, degp_ref, w_ref, b_ref, o_ref):
  deg = degp_ref[0, 0:n, 0:1] + degp_ref[1, 0:n, 0:1] + 1.0
  z = ((vp_ref[0, 0:n, :] + vp_ref[1, 0:n, :] + u_ref[0:n, :])
       * jax.lax.rsqrt(deg))
  o_ref[...] = (
      jnp.dot(z, w_ref[...], preferred_element_type=jnp.float32) + b_ref[...]
  )


@jax.jit
def kernel(x, edge_index, W, b):
  n, d = x.shape
  e = edge_index.shape[1]
  _, np_rows = _padded_rows(n)

  grain = NW * CHUNK * max(NBUF, DEG_GRP)
  e_pad = -(-e // grain) * grain
  src = edge_index[0].astype(jnp.int32)
  dst = edge_index[1].astype(jnp.int32)
  if e_pad != e:
    fill = jnp.full((e_pad - e,), n, jnp.int32)
    src = jnp.concatenate([src, fill])
    dst = jnp.concatenate([dst, fill])
  n_chunks = e_pad // CHUNK
  src2 = src.reshape(n_chunks, CHUNK)
  dst2 = dst.reshape(n_chunks, CHUNK)
  dst3 = dst.reshape(n_chunks, 1, CHUNK)

  degp = _make_degree_kernel(n, n_chunks)(dst2)
  u = pl.pallas_call(
      functools.partial(_scale_body, n, np_rows),
      out_shape=jax.ShapeDtypeStruct((np_rows, d), jnp.float32),
  )(degp, x)
  vp = _make_scatter_kernel(n, n_chunks, d)(u, src2, dst3)
  out = pl.pallas_call(
      functools.partial(_combine_body, n),
      out_shape=jax.ShapeDtypeStruct((n, d), jnp.float32),
  )(vp, u, degp, W, b.reshape(1, d))
  return out


# speedup vs baseline: 2.5422x; 2.5422x over previous
import functools

import jax
import jax.numpy as jnp
from jax.experimental import pallas as pl
from jax.experimental.pallas import tpu as pltpu
from jax.experimental.pallas import tpu_sc as plsc

NC = 2
NS = 16
NW = NC * NS
LANES = 16
CHUNK = 128
NBUF = 2
DEG_GRP = 8


def _padded_rows(n_nodes):
  per = -(-n_nodes // (NS * 8)) * 8
  return per, per * NS


def _zero_shared_rows(stage_v, sh, rbase, rows, width):
  @pl.loop(0, CHUNK)
  def _(i):
    @pl.loop(0, width, step=LANES)
    def _(jj):
      stage_v[i, pl.ds(jj, LANES)] = jnp.zeros((LANES,), jnp.float32)

  @pl.loop(0, rows // CHUNK)
  def _(t):
    pltpu.sync_copy(stage_v, sh.at[pl.ds(rbase + t * CHUNK, CHUNK)])
  rem = rows % CHUNK
  if rem:
    pltpu.sync_copy(stage_v.at[pl.ds(0, rem)],
                    sh.at[pl.ds(rbase + (rows // CHUNK) * CHUNK, rem)])


def _make_degree_kernel(n_nodes, n_chunks):
  k_tile = n_chunks // NW
  rows_tile, np_rows = _padded_rows(n_nodes)

  mesh = plsc.VectorSubcoreMesh(core_axis_name="c", subcore_axis_name="s",
                                num_cores=NC, num_subcores=NS)

  @functools.partial(
      pl.kernel,
      out_type=jax.ShapeDtypeStruct((NC, np_rows, LANES), jnp.float32),
      mesh=mesh,
      scratch_types=[
          pltpu.VMEM((k_tile, CHUNK), jnp.int32),
          pltpu.VMEM((CHUNK, LANES), jnp.float32),
          pltpu.VMEM_SHARED((np_rows, LANES), jnp.float32),
      ] + [pltpu.SemaphoreType.DMA] * DEG_GRP,
  )
  def k(dst_hbm, degp_hbm, dst_v, ones_v, deg_sh, *sems):
    c = jax.lax.axis_index("c")
    s = jax.lax.axis_index("s")
    wid = c * NS + s
    rbase = s * rows_tile

    pltpu.sync_copy(dst_hbm.at[pl.ds(wid * k_tile, k_tile)], dst_v)
    _zero_shared_rows(ones_v, deg_sh, rbase, rows_tile, LANES)

    @pl.loop(0, CHUNK)
    def _(i):
      ones_v[i, :] = jnp.ones((LANES,), jnp.float32)

    plsc.subcore_barrier()

    @pl.loop(0, k_tile, step=DEG_GRP)
    def _(t):
      descs = []
      for b in range(DEG_GRP):
        descs.append(pltpu.async_copy(
            ones_v, deg_sh.at[dst_v.at[t + b]], sems[b], add=True))
      for d_ in descs:
        d_.wait()

    plsc.subcore_barrier()
    pltpu.sync_copy(deg_sh.at[pl.ds(rbase, rows_tile)],
                    degp_hbm.at[c].at[pl.ds(rbase, rows_tile)])

  return k


def _make_scatter_kernel(n_nodes, n_chunks, d):
  k_tile = n_chunks // NW
  rows_tile, np_rows = _padded_rows(n_nodes)
  ngroups = k_tile // NBUF
  assert k_tile % NBUF == 0

  mesh = plsc.VectorSubcoreMesh(core_axis_name="c", subcore_axis_name="s",
                                num_cores=NC, num_subcores=NS)

  rows_scratch = [pltpu.VMEM((CHUNK, d), jnp.float32) for _ in range(NBUF)]
  dstb_scratch = [pltpu.VMEM((1, CHUNK), jnp.int32) for _ in range(NBUF)]

  @functools.partial(
      pl.kernel,
      out_type=jax.ShapeDtypeStruct((NC, np_rows, d), jnp.float32),
      mesh=mesh,
      scratch_types=[
          pltpu.VMEM((k_tile, CHUNK), jnp.int32),
          pltpu.VMEM_SHARED((np_rows, d), jnp.float32),
      ] + rows_scratch + dstb_scratch
        + [pltpu.SemaphoreType.DMA] * (2 * NBUF),
  )
  def k(u_hbm, src_hbm, dst_hbm, vp_hbm, src_v, v_sh, *rest):
    rows = rest[:NBUF]
    dstb = rest[NBUF:2 * NBUF]
    gsem = rest[2 * NBUF:3 * NBUF]
    dsem = rest[3 * NBUF:4 * NBUF]
    c = jax.lax.axis_index("c")
    s = jax.lax.axis_index("s")
    wid = c * NS + s
    rbase = s * rows_tile
    crow = wid * k_tile

    pltpu.sync_copy(src_hbm.at[pl.ds(crow, k_tile)], src_v)

    _zero_shared_rows(rows[0], v_sh, rbase, rows_tile, d)

    for b in range(NBUF):
      pltpu.async_copy(dst_hbm.at[crow + b], dstb[b], dsem[b])
      pltpu.async_copy(u_hbm.at[src_v.at[b]], rows[b], gsem[b])

    plsc.subcore_barrier()

    def wait_idx(b):
      pltpu.make_async_copy(dst_hbm.at[0], dstb[b], dsem[b]).wait()

    def wait_rows(b):
      pltpu.make_async_copy(u_hbm.at[pl.ds(0, CHUNK)], rows[b], gsem[b]).wait()

    @pl.loop(0, ngroups - 1)
    def _(t):
      for b in range(NBUF):
        j = t * NBUF + b
        wait_idx(b)
        wait_rows(b)
        pltpu.sync_copy(rows[b], v_sh.at[dstb[b].at[0]], add=True)
        pltpu.async_copy(dst_hbm.at[crow + j + NBUF], dstb[b], dsem[b])
        pltpu.async_copy(u_hbm.at[src_v.at[j + NBUF]], rows[b], gsem[b])

    for b in range(NBUF):
      wait_idx(b)
      wait_rows(b)
      pltpu.sync_copy(rows[b], v_sh.at[dstb[b].at[0]], add=True)

    plsc.subcore_barrier()
    pltpu.sync_copy(v_sh.at[pl.ds(rbase, rows_tile)],
                    vp_hbm.at[c].at[pl.ds(rbase, rows_tile)])

  return k


def _scale_body(n, np_rows, degp_ref, x_ref, u_ref):
  deg = degp_ref[0, 0:n, 0:1] + degp_ref[1, 0:n, 0:1] + 1.0
  u_ref[0:n, :] = x_ref[...] * jax.lax.rsqrt(deg)
  u_ref[n:np_rows, :] = jnp.zeros((np_rows - n, x_ref.shape[1]), jnp.float32)


def _combine_body(n, vp_ref, u_ref, degp_ref, w_ref, b_ref, o_ref):
  deg = degp_ref[0, 0:n, 0:1] + degp_ref[1, 0:n, 0:1] + 1.0
  z = ((vp_ref[0, 0:n, :] + vp_ref[1, 0:n, :] + u_ref[0:n, :])
       * jax.lax.rsqrt(deg))
  o_ref[...] = (
      jnp.dot(z, w_ref[...], preferred_element_type=jnp.float32) + b_ref[...]
  )


@jax.jit
def kernel(x, edge_index, W, b):
  n, d = x.shape
  e = edge_index.shape[1]
  _, np_rows = _padded_rows(n)

  grain = NW * CHUNK * max(NBUF, DEG_GRP)
  e_pad = -(-e // grain) * grain
  src = edge_index[0].astype(jnp.int32)
  dst = edge_index[1].astype(jnp.int32)
  if e_pad != e:
    fill = n + (jnp.arange(e_pad - e, dtype=jnp.int32) % (np_rows - n))
    src = jnp.concatenate([src, fill])
    dst = jnp.concatenate([dst, fill])
  n_chunks = e_pad // CHUNK
  src2 = src.reshape(n_chunks, CHUNK)
  dst2 = dst.reshape(n_chunks, CHUNK)
  dst3 = dst.reshape(n_chunks, 1, CHUNK)

  degp = _make_degree_kernel(n, n_chunks)(dst2)
  u = pl.pallas_call(
      functools.partial(_scale_body, n, np_rows),
      out_shape=jax.ShapeDtypeStruct((np_rows, d), jnp.float32),
  )(degp, x)
  vp = _make_scatter_kernel(n, n_chunks, d)(u, src2, dst3)
  out = pl.pallas_call(
      functools.partial(_combine_body, n),
      out_shape=jax.ShapeDtypeStruct((n, d), jnp.float32),
  )(vp, u, degp, W, b.reshape(1, d))
  return out
